# baseline (speedup 1.0000x reference)
import functools

import jax
import jax.numpy as jnp
import numpy as np
from jax import lax
from jax.experimental import pallas as pl
from jax.experimental.pallas import tpu as pltpu
from jax.experimental.pallas import tpu_sc as plsc

D = 256
NH = 8
NL = 4
NP = 4
HD = D // NH
NTAP = 4
_SHAPES = [(128, 128), (64, 64), (32, 32), (16, 16)]
LQ = sum(h * w for h, w in _SHAPES)
B = 2
BLK = 640
NBLK = LQ // BLK
E = NH * NL * NP * NTAP
ROWS = B * LQ * NH

_lane = np.arange(128)
_lane_l = (_lane // 4) % 4
_lane_h = _lane // 16
_W_l = np.array([_SHAPES[l][1] for l in _lane_l], np.float32)
_H_l = np.array([_SHAPES[l][0] for l in _lane_l], np.float32)
_lsi_np = np.cumsum([0] + [h * w for h, w in _SHAPES])[:-1]
_LSI_H = (_lsi_np[_lane_l] * NH + _lane_h).astype(np.int32)
_Sx = np.zeros((NL, 128), np.float32)
_Sx[_lane_l, _lane] = _W_l
_Sy = np.zeros((NL, 128), np.float32)
_Sy[_lane_l, _lane] = _H_l
_G = (_lane_h[:, None] == _lane_h[None, :]).astype(np.float32)
_Plo = np.zeros((D, 128), np.float32)
_Phi = np.zeros((D, 128), np.float32)
for _j in range(128):
    _Plo[(_j // 16) * HD + (_j % 16), _j] = 1.0
    _Phi[(_j // 16) * HD + 16 + (_j % 16), _j] = 1.0


def _stage1_body(q_ref, v_ref, wv_ref, bv_ref, wa_ref, ba_ref, wox_ref,
                 woy_ref, bx_ref, by_ref, sx_ref, sy_ref, wl_ref, hl_ref,
                 lsih_ref, g_ref, plo_ref, phi_ref, rpx_ref, rpy_ref,
                 val_ref, idx_ref, w_ref):
    b = pl.program_id(0)
    q = q_ref[0]
    v = v_ref[0]
    valf = jnp.dot(v, wv_ref[...], preferred_element_type=jnp.float32,
                   precision=lax.Precision.HIGHEST) + bv_ref[...]

    def bf16_bits(x):
        bits = lax.bitcast_convert_type(x, jnp.int32)
        rnd = bits + 0x7FFF + jnp.bitwise_and(
            lax.shift_right_logical(bits, 16), 1)
        return lax.shift_right_logical(rnd, 16)

    lo = jnp.dot(valf, plo_ref[...], preferred_element_type=jnp.float32,
                 precision=lax.Precision.HIGHEST)
    hi = jnp.dot(valf, phi_ref[...], preferred_element_type=jnp.float32,
                 precision=lax.Precision.HIGHEST)
    val_ref[0] = jnp.bitwise_or(bf16_bits(lo),
                                lax.shift_left(bf16_bits(hi), 16))
    logits = jnp.dot(q, wa_ref[...], preferred_element_type=jnp.float32, precision=lax.Precision.HIGHEST) + ba_ref[...]
    m = jnp.max(logits, axis=1, keepdims=True)
    e = jnp.exp(logits - m)
    den = jnp.dot(e, g_ref[...], preferred_element_type=jnp.float32, precision=lax.Precision.HIGHEST)
    att = e / den
    x = (jnp.dot(q, wox_ref[...], preferred_element_type=jnp.float32, precision=lax.Precision.HIGHEST)
         + jnp.dot(rpx_ref[0], sx_ref[...], preferred_element_type=jnp.float32, precision=lax.Precision.HIGHEST)
         + bx_ref[...])
    y = (jnp.dot(q, woy_ref[...], preferred_element_type=jnp.float32, precision=lax.Precision.HIGHEST)
         + jnp.dot(rpy_ref[0], sy_ref[...], preferred_element_type=jnp.float32, precision=lax.Precision.HIGHEST)
         + by_ref[...])
    wl = wl_ref[...]
    hl = hl_ref[...]
    x0 = jnp.floor(x)
    y0 = jnp.floor(y)
    fx1 = x - x0
    fx0 = 1.0 - fx1
    fy1 = y - y0
    fy0 = 1.0 - fy1
    base_b = b * (LQ * NH)
    lsih = lsih_ref[...]
    for t, (dx, dy) in enumerate(((0, 0), (1, 0), (0, 1), (1, 1))):
        xt = x0 + float(dx)
        yt = y0 + float(dy)
        valid = ((xt >= 0.0) & (xt <= wl - 1.0)
                 & (yt >= 0.0) & (yt <= hl - 1.0))
        xc = jnp.clip(xt, 0.0, wl - 1.0).astype(jnp.int32)
        yc = jnp.clip(yt, 0.0, hl - 1.0).astype(jnp.int32)
        spatial = yc * wl.astype(jnp.int32) + xc
        idx_ref[0, :, t * 128:(t + 1) * 128] = base_b + lsih + spatial * NH
        wt = (fx0 if dx == 0 else fx1) * (fy0 if dy == 0 else fy1)
        w_ref[0, :, t * 128:(t + 1) * 128] = att * wt * valid.astype(jnp.float32)


def _stage1(query, value, rpx, rpy, W_val, b_val, W_attn, b_attn,
            W_offx, W_offy, bx, by):
    consts = [
        jnp.asarray(_Sx), jnp.asarray(_Sy),
        jnp.asarray(_W_l).reshape(1, 128), jnp.asarray(_H_l).reshape(1, 128),
        jnp.asarray(_LSI_H).reshape(1, 128), jnp.asarray(_G),
        jnp.asarray(_Plo), jnp.asarray(_Phi),
    ]

    def whole(shape):
        return pl.BlockSpec(shape, lambda b, i: tuple(0 for _ in shape))

    return pl.pallas_call(
        _stage1_body,
        grid=(B, NBLK),
        in_specs=[
            pl.BlockSpec((1, BLK, D), lambda b, i: (b, i, 0)),
            pl.BlockSpec((1, BLK, D), lambda b, i: (b, i, 0)),
            whole((D, D)), whole((1, D)),
            whole((D, 128)), whole((1, 128)),
            whole((D, 128)), whole((D, 128)),
            whole((1, 128)), whole((1, 128)),
            whole((NL, 128)), whole((NL, 128)),
            whole((1, 128)), whole((1, 128)),
            whole((1, 128)), whole((128, 128)),
            whole((D, 128)), whole((D, 128)),
            pl.BlockSpec((1, BLK, NL), lambda b, i: (b, i, 0)),
            pl.BlockSpec((1, BLK, NL), lambda b, i: (b, i, 0)),
        ],
        out_specs=[
            pl.BlockSpec((1, BLK, 128), lambda b, i: (b, i, 0)),
            pl.BlockSpec((1, BLK, E), lambda b, i: (b, i, 0)),
            pl.BlockSpec((1, BLK, E), lambda b, i: (b, i, 0)),
        ],
        out_shape=[
            jax.ShapeDtypeStruct((B, LQ, 128), jnp.int32),
            jax.ShapeDtypeStruct((B, LQ, E), jnp.int32),
            jax.ShapeDtypeStruct((B, LQ, E), jnp.float32),
        ],
        compiler_params=pltpu.CompilerParams(
            dimension_semantics=("parallel", "parallel")),
    )(query, value, W_val, b_val, W_attn, b_attn, W_offx, W_offy, bx, by,
      *consts, rpx, rpy)


QC = 2
NQ = B * LQ
NGATH = E // 128


def _sc_body(tab_hbm, idx_hbm, w_hbm, out_hbm,
             idx_v, w_v, rows_v, out_v, gsem, osem):
    nc = 2
    wid = lax.axis_index("s") * nc + lax.axis_index("c")
    per_tile = NQ // 32
    iters = per_tile // QC
    qbase = wid * per_tile

    def fetch(q0, s):
        pltpu.sync_copy(idx_hbm.at[pl.ds(q0 * NGATH, QC * NGATH)], idx_v.at[s])
        pltpu.sync_copy(w_hbm.at[pl.ds(q0 * E, QC * E)], w_v.at[s])
        for j in range(QC * NGATH):
            pltpu.async_copy(
                tab_hbm.at[idx_v.at[s, j]],
                rows_v.at[s, pl.ds(j * 128, 128)], gsem)

    def drain(s):
        for j in range(QC * NGATH):
            pltpu.make_async_copy(
                tab_hbm.at[idx_v.at[s, j]],
                rows_v.at[s, pl.ds(j * 128, 128)], gsem).wait()

    def wait_out(s):
        pltpu.make_async_copy(
            out_v.at[s], out_hbm.at[pl.ds(0, QC * NH)], osem).wait()

    def compute(q0, s):
        def qh_body(qh, c2):
            rbase = (lax.shift_right_logical(qh, 3) * E
                     + jnp.bitwise_and(qh, 7) * 16)
            z = jnp.zeros((16,), jnp.float32)
            p0 = []
            p1 = []
            for t in range(NTAP):
                wvec = w_v[s, pl.ds(rbase + t * 128, 16)]
                u0, u1, v0, v1 = z, z, z, z
                for k in range(16):
                    wk = wvec[k]
                    r = rbase + t * 128 + k
                    row = rows_v[s, r, 0:16]
                    ev = lax.bitcast_convert_type(
                        lax.shift_left(row, 16), jnp.float32)
                    od = lax.bitcast_convert_type(
                        jnp.bitwise_and(row, -65536), jnp.float32)
                    if k & 1:
                        v0 = v0 + wk * ev
                        v1 = v1 + wk * od
                    else:
                        u0 = u0 + wk * ev
                        u1 = u1 + wk * od
                p0.append(u0 + v0)
                p1.append(u1 + v1)
            out_v[s, qh, 0:16] = (p0[0] + p0[1]) + (p0[2] + p0[3])
            out_v[s, qh, 16:32] = (p1[0] + p1[1]) + (p1[2] + p1[3])
            return c2

        lax.fori_loop(0, QC * NH, qh_body, 0)
        pltpu.async_copy(out_v.at[s], out_hbm.at[pl.ds(q0 * NH, QC * NH)], osem)

    npairs = iters // 2
    fetch(qbase, 0)

    def pair_body(p, carry):
        q0 = qbase + p * 2 * QC

        fetch(q0 + QC, 1)
        drain(0)

        @pl.when(p >= 1)
        def _():
            wait_out(0)

        compute(q0, 0)

        @pl.when(p + 1 < npairs)
        def _():
            fetch(q0 + 2 * QC, 0)

        drain(1)

        @pl.when(p >= 1)
        def _():
            wait_out(1)

        compute(q0 + QC, 1)
        return carry

    lax.fori_loop(0, npairs, pair_body, 0)
    wait_out(0)
    wait_out(1)


def _combine_sc(tab, idx4, w2):
    mesh = plsc.VectorSubcoreMesh(core_axis_name="c", subcore_axis_name="s")
    run = functools.partial(
        pl.kernel,
        mesh=mesh,
        out_type=jax.ShapeDtypeStruct((ROWS, HD), jnp.float32),
        name="msdeform_combine",
        scratch_types=[
            pltpu.VMEM((2, QC * NGATH, 128), jnp.int32),
            pltpu.VMEM((2, QC * E), jnp.float32),
            pltpu.VMEM((2, QC * E, HD // 2), jnp.int32),
            pltpu.VMEM((2, QC * NH, HD), jnp.float32),
            pltpu.SemaphoreType.DMA,
            pltpu.SemaphoreType.DMA,
        ],
        compiler_params=pltpu.CompilerParams(use_tc_tiling_on_sc=False),
    )(_sc_body)
    return run(tab, idx4, w2)


def _stage2_body(x_ref, w_ref, b_ref, o_ref):
    o_ref[0] = (jnp.dot(x_ref[0], w_ref[...], preferred_element_type=jnp.float32, precision=lax.Precision.HIGHEST)
                + b_ref[...])


def _stage2(x, W_out, b_out):
    return pl.pallas_call(
        _stage2_body,
        grid=(B, NBLK),
        in_specs=[
            pl.BlockSpec((1, BLK, D), lambda b, i: (b, i, 0)),
            pl.BlockSpec((D, D), lambda b, i: (0, 0)),
            pl.BlockSpec((1, D), lambda b, i: (0, 0)),
        ],
        out_specs=pl.BlockSpec((1, BLK, D), lambda b, i: (b, i, 0)),
        out_shape=jax.ShapeDtypeStruct((B, LQ, D), jnp.float32),
        compiler_params=pltpu.CompilerParams(
            dimension_semantics=("parallel", "parallel")),
    )(x, W_out, b_out)


_ATT_PERM = np.array([h * 16 + p * NL + l
                      for h in range(NH) for l in range(NL)
                      for p in range(NP)], np.int32)


def kernel(query, reference_points, value, spatial_shapes, level_start_index,
           W_off, b_off, W_attn, b_attn, W_val, b_val, W_out, b_out):
    W_attn = W_attn[:, _ATT_PERM]
    b_attn = b_attn[_ATT_PERM]
    rpx = reference_points[..., 0]
    rpy = reference_points[..., 1]
    W_offx = W_off[:, 0::2]
    W_offy = W_off[:, 1::2]
    bx = (b_off[0::2] - 0.5).reshape(1, 128)
    by = (b_off[1::2] - 0.5).reshape(1, 128)
    val, idxs, ws = _stage1(
        query, value, rpx, rpy, W_val, b_val.reshape(1, D),
        W_attn, b_attn.reshape(1, 128), W_offx, W_offy, bx, by)
    tab = val.reshape(ROWS, HD // 2)
    idx4 = idxs.reshape(NQ * NGATH, 128)
    w2 = ws.reshape(NQ * E)
    out1 = _combine_sc(tab, idx4, w2).reshape(B, LQ, D)
    return _stage2(out1, W_out, b_out.reshape(1, D))


# speedup vs baseline: 1.0528x; 1.0528x over previous
import functools

import jax
import jax.numpy as jnp
import numpy as np
from jax import lax
from jax.experimental import pallas as pl
from jax.experimental.pallas import tpu as pltpu
from jax.experimental.pallas import tpu_sc as plsc

D = 256
NH = 8
NL = 4
NP = 4
HD = D // NH
NTAP = 4
_SHAPES = [(128, 128), (64, 64), (32, 32), (16, 16)]
LQ = sum(h * w for h, w in _SHAPES)
B = 2
BLK = 640
NBLK = LQ // BLK
E = NH * NL * NP * NTAP
ROWS = B * LQ * NH

_lane = np.arange(128)
_lane_l = (_lane // 4) % 4
_lane_h = _lane // 16
_W_l = np.array([_SHAPES[l][1] for l in _lane_l], np.float32)
_H_l = np.array([_SHAPES[l][0] for l in _lane_l], np.float32)
_lsi_np = np.cumsum([0] + [h * w for h, w in _SHAPES])[:-1]
_LSI_H = (_lsi_np[_lane_l] * NH + _lane_h).astype(np.int32)
_Sx = np.zeros((NL, 128), np.float32)
_Sx[_lane_l, _lane] = _W_l
_Sy = np.zeros((NL, 128), np.float32)
_Sy[_lane_l, _lane] = _H_l
_G = (_lane_h[:, None] == _lane_h[None, :]).astype(np.float32)
_Plo = np.zeros((D, 128), np.float32)
_Phi = np.zeros((D, 128), np.float32)
for _j in range(128):
    _Plo[(_j // 16) * HD + (_j % 16), _j] = 1.0
    _Phi[(_j // 16) * HD + 16 + (_j % 16), _j] = 1.0


def _stage1_body(q_ref, v_ref, wv_ref, bv_ref, wa_ref, ba_ref, wox_ref,
                 woy_ref, bx_ref, by_ref, sx_ref, sy_ref, wl_ref, hl_ref,
                 lsih_ref, g_ref, plo_ref, phi_ref, rpx_ref, rpy_ref,
                 val_ref, idx_ref, w_ref):
    b = pl.program_id(0)
    q = q_ref[0]
    v = v_ref[0]
    valf = jnp.dot(v, wv_ref[...], preferred_element_type=jnp.float32,
                   precision=lax.Precision.HIGHEST) + bv_ref[...]

    def bf16_bits(x):
        bits = lax.bitcast_convert_type(x, jnp.int32)
        rnd = bits + 0x7FFF + jnp.bitwise_and(
            lax.shift_right_logical(bits, 16), 1)
        return lax.shift_right_logical(rnd, 16)

    lo = jnp.dot(valf, plo_ref[...], preferred_element_type=jnp.float32,
                 precision=lax.Precision.HIGHEST)
    hi = jnp.dot(valf, phi_ref[...], preferred_element_type=jnp.float32,
                 precision=lax.Precision.HIGHEST)
    val_ref[0] = jnp.bitwise_or(bf16_bits(lo),
                                lax.shift_left(bf16_bits(hi), 16))
    logits = jnp.dot(q, wa_ref[...], preferred_element_type=jnp.float32, precision=lax.Precision.HIGHEST) + ba_ref[...]
    m = jnp.max(logits, axis=1, keepdims=True)
    e = jnp.exp(logits - m)
    den = jnp.dot(e, g_ref[...], preferred_element_type=jnp.float32, precision=lax.Precision.HIGHEST)
    att = e / den
    x = (jnp.dot(q, wox_ref[...], preferred_element_type=jnp.float32, precision=lax.Precision.HIGHEST)
         + jnp.dot(rpx_ref[0], sx_ref[...], preferred_element_type=jnp.float32, precision=lax.Precision.HIGHEST)
         + bx_ref[...])
    y = (jnp.dot(q, woy_ref[...], preferred_element_type=jnp.float32, precision=lax.Precision.HIGHEST)
         + jnp.dot(rpy_ref[0], sy_ref[...], preferred_element_type=jnp.float32, precision=lax.Precision.HIGHEST)
         + by_ref[...])
    wl = wl_ref[...]
    hl = hl_ref[...]
    x0 = jnp.floor(x)
    y0 = jnp.floor(y)
    fx1 = x - x0
    fx0 = 1.0 - fx1
    fy1 = y - y0
    fy0 = 1.0 - fy1
    base_b = b * (LQ * NH)
    lsih = lsih_ref[...]
    for t, (dx, dy) in enumerate(((0, 0), (1, 0), (0, 1), (1, 1))):
        xt = x0 + float(dx)
        yt = y0 + float(dy)
        valid = ((xt >= 0.0) & (xt <= wl - 1.0)
                 & (yt >= 0.0) & (yt <= hl - 1.0))
        xc = jnp.clip(xt, 0.0, wl - 1.0).astype(jnp.int32)
        yc = jnp.clip(yt, 0.0, hl - 1.0).astype(jnp.int32)
        spatial = yc * wl.astype(jnp.int32) + xc
        idx_ref[0, :, t * 128:(t + 1) * 128] = base_b + lsih + spatial * NH
        wt = (fx0 if dx == 0 else fx1) * (fy0 if dy == 0 else fy1)
        w_ref[0, :, t * 128:(t + 1) * 128] = att * wt * valid.astype(jnp.float32)


def _stage1(query, value, rpx, rpy, W_val, b_val, W_attn, b_attn,
            W_offx, W_offy, bx, by):
    consts = [
        jnp.asarray(_Sx), jnp.asarray(_Sy),
        jnp.asarray(_W_l).reshape(1, 128), jnp.asarray(_H_l).reshape(1, 128),
        jnp.asarray(_LSI_H).reshape(1, 128), jnp.asarray(_G),
        jnp.asarray(_Plo), jnp.asarray(_Phi),
    ]

    def whole(shape):
        return pl.BlockSpec(shape, lambda b, i: tuple(0 for _ in shape))

    return pl.pallas_call(
        _stage1_body,
        grid=(B, NBLK),
        in_specs=[
            pl.BlockSpec((1, BLK, D), lambda b, i: (b, i, 0)),
            pl.BlockSpec((1, BLK, D), lambda b, i: (b, i, 0)),
            whole((D, D)), whole((1, D)),
            whole((D, 128)), whole((1, 128)),
            whole((D, 128)), whole((D, 128)),
            whole((1, 128)), whole((1, 128)),
            whole((NL, 128)), whole((NL, 128)),
            whole((1, 128)), whole((1, 128)),
            whole((1, 128)), whole((128, 128)),
            whole((D, 128)), whole((D, 128)),
            pl.BlockSpec((1, BLK, NL), lambda b, i: (b, i, 0)),
            pl.BlockSpec((1, BLK, NL), lambda b, i: (b, i, 0)),
        ],
        out_specs=[
            pl.BlockSpec((1, BLK, 128), lambda b, i: (b, i, 0)),
            pl.BlockSpec((1, BLK, E), lambda b, i: (b, i, 0)),
            pl.BlockSpec((1, BLK, E), lambda b, i: (b, i, 0)),
        ],
        out_shape=[
            jax.ShapeDtypeStruct((B, LQ, 128), jnp.int32),
            jax.ShapeDtypeStruct((B, LQ, E), jnp.int32),
            jax.ShapeDtypeStruct((B, LQ, E), jnp.float32),
        ],
        compiler_params=pltpu.CompilerParams(
            dimension_semantics=("parallel", "parallel")),
    )(query, value, W_val, b_val, W_attn, b_attn, W_offx, W_offy, bx, by,
      *consts, rpx, rpy)


QC = 2
NQ = B * LQ
NGATH = E // 128


def _sc_body(tab_hbm, idx_hbm, w_hbm, out_hbm,
             idx_v, w_v, rows_v, out_v, gsem, osem):
    nc = 2
    wid = lax.axis_index("s") * nc + lax.axis_index("c")
    per_tile = NQ // 32
    iters = per_tile // QC
    qbase = wid * per_tile

    def fetch(q0, s):
        pltpu.sync_copy(idx_hbm.at[pl.ds(q0 * NGATH, QC * NGATH)], idx_v.at[s])
        pltpu.sync_copy(w_hbm.at[pl.ds(q0 * E, QC * E)], w_v.at[s])
        for j in range(QC * NGATH):
            pltpu.async_copy(
                tab_hbm.at[idx_v.at[s, j]],
                rows_v.at[s, pl.ds(j * 128, 128)], gsem)

    def drain(s):
        for j in range(QC * NGATH):
            pltpu.make_async_copy(
                tab_hbm.at[idx_v.at[s, j]],
                rows_v.at[s, pl.ds(j * 128, 128)], gsem).wait()

    def wait_out(s):
        pltpu.make_async_copy(
            out_v.at[s], out_hbm.at[pl.ds(0, QC * NH)], osem).wait()

    def compute(q0, s):
        def qh_body(qh, c2):
            rbase = (lax.shift_right_logical(qh, 3) * E
                     + jnp.bitwise_and(qh, 7) * 16)
            z = jnp.zeros((16,), jnp.float32)
            p0 = []
            p1 = []
            for t in range(NTAP):
                wvec = w_v[s, pl.ds(rbase + t * 128, 16)]
                u0, u1, v0, v1 = z, z, z, z
                for k in range(16):
                    wk = wvec[k]
                    r = rbase + t * 128 + k
                    row = rows_v[s, r, 0:16]
                    ev = lax.bitcast_convert_type(
                        lax.shift_left(row, 16), jnp.float32)
                    od = lax.bitcast_convert_type(row, jnp.float32)
                    if k & 1:
                        v0 = v0 + wk * ev
                        v1 = v1 + wk * od
                    else:
                        u0 = u0 + wk * ev
                        u1 = u1 + wk * od
                p0.append(u0 + v0)
                p1.append(u1 + v1)
            out_v[s, qh, 0:16] = (p0[0] + p0[1]) + (p0[2] + p0[3])
            out_v[s, qh, 16:32] = (p1[0] + p1[1]) + (p1[2] + p1[3])
            return c2

        lax.fori_loop(0, QC * NH, qh_body, 0)
        pltpu.async_copy(out_v.at[s], out_hbm.at[pl.ds(q0 * NH, QC * NH)], osem)

    npairs = iters // 2
    fetch(qbase, 0)

    def pair_body(p, carry):
        q0 = qbase + p * 2 * QC

        fetch(q0 + QC, 1)
        drain(0)

        @pl.when(p >= 1)
        def _():
            wait_out(0)

        compute(q0, 0)

        @pl.when(p + 1 < npairs)
        def _():
            fetch(q0 + 2 * QC, 0)

        drain(1)

        @pl.when(p >= 1)
        def _():
            wait_out(1)

        compute(q0 + QC, 1)
        return carry

    lax.fori_loop(0, npairs, pair_body, 0)
    wait_out(0)
    wait_out(1)


def _combine_sc(tab, idx4, w2):
    mesh = plsc.VectorSubcoreMesh(core_axis_name="c", subcore_axis_name="s")
    run = functools.partial(
        pl.kernel,
        mesh=mesh,
        out_type=jax.ShapeDtypeStruct((ROWS, HD), jnp.float32),
        name="msdeform_combine",
        scratch_types=[
            pltpu.VMEM((2, QC * NGATH, 128), jnp.int32),
            pltpu.VMEM((2, QC * E), jnp.float32),
            pltpu.VMEM((2, QC * E, HD // 2), jnp.int32),
            pltpu.VMEM((2, QC * NH, HD), jnp.float32),
            pltpu.SemaphoreType.DMA,
            pltpu.SemaphoreType.DMA,
        ],
        compiler_params=pltpu.CompilerParams(use_tc_tiling_on_sc=False),
    )(_sc_body)
    return run(tab, idx4, w2)


def _stage2_body(x_ref, w_ref, b_ref, o_ref):
    o_ref[0] = (jnp.dot(x_ref[0], w_ref[...], preferred_element_type=jnp.float32, precision=lax.Precision.HIGHEST)
                + b_ref[...])


def _stage2(x, W_out, b_out):
    return pl.pallas_call(
        _stage2_body,
        grid=(B, NBLK),
        in_specs=[
            pl.BlockSpec((1, BLK, D), lambda b, i: (b, i, 0)),
            pl.BlockSpec((D, D), lambda b, i: (0, 0)),
            pl.BlockSpec((1, D), lambda b, i: (0, 0)),
        ],
        out_specs=pl.BlockSpec((1, BLK, D), lambda b, i: (b, i, 0)),
        out_shape=jax.ShapeDtypeStruct((B, LQ, D), jnp.float32),
        compiler_params=pltpu.CompilerParams(
            dimension_semantics=("parallel", "parallel")),
    )(x, W_out, b_out)


_ATT_PERM = np.array([h * 16 + p * NL + l
                      for h in range(NH) for l in range(NL)
                      for p in range(NP)], np.int32)


def kernel(query, reference_points, value, spatial_shapes, level_start_index,
           W_off, b_off, W_attn, b_attn, W_val, b_val, W_out, b_out):
    W_attn = W_attn[:, _ATT_PERM]
    b_attn = b_attn[_ATT_PERM]
    rpx = reference_points[..., 0]
    rpy = reference_points[..., 1]
    W_offx = W_off[:, 0::2]
    W_offy = W_off[:, 1::2]
    bx = (b_off[0::2] - 0.5).reshape(1, 128)
    by = (b_off[1::2] - 0.5).reshape(1, 128)
    val, idxs, ws = _stage1(
        query, value, rpx, rpy, W_val, b_val.reshape(1, D),
        W_attn, b_attn.reshape(1, 128), W_offx, W_offy, bx, by)
    tab = val.reshape(ROWS, HD // 2)
    idx4 = idxs.reshape(NQ * NGATH, 128)
    w2 = ws.reshape(NQ * E)
    out1 = _combine_sc(tab, idx4, w2).reshape(B, LQ, D)
    return _stage2(out1, W_out, b_out.reshape(1, D))


# speedup vs baseline: 1.2297x; 1.1680x over previous
import functools

import jax
import jax.numpy as jnp
import numpy as np
from jax import lax
from jax.experimental import pallas as pl
from jax.experimental.pallas import tpu as pltpu
from jax.experimental.pallas import tpu_sc as plsc

D = 256
NH = 8
NL = 4
NP = 4
HD = D // NH
NTAP = 4
_SHAPES = [(128, 128), (64, 64), (32, 32), (16, 16)]
LQ = sum(h * w for h, w in _SHAPES)
B = 2
BLK = 640
NBLK = LQ // BLK
E = NH * NL * NP * NTAP
ROWS = B * LQ * NH

_lane = np.arange(128)
_lane_l = (_lane // 4) % 4
_lane_h = _lane // 16
_W_l = np.array([_SHAPES[l][1] for l in _lane_l], np.float32)
_H_l = np.array([_SHAPES[l][0] for l in _lane_l], np.float32)
_lsi_np = np.cumsum([0] + [h * w for h, w in _SHAPES])[:-1]
_LSI_H = (_lsi_np[_lane_l] * NH + _lane_h).astype(np.int32)
_Sx = np.zeros((NL, 128), np.float32)
_Sx[_lane_l, _lane] = _W_l
_Sy = np.zeros((NL, 128), np.float32)
_Sy[_lane_l, _lane] = _H_l
_G = (_lane_h[:, None] == _lane_h[None, :]).astype(np.float32)
_Plo = np.zeros((D, 128), np.float32)
_Phi = np.zeros((D, 128), np.float32)
for _j in range(128):
    _Plo[(_j // 16) * HD + (_j % 16), _j] = 1.0
    _Phi[(_j // 16) * HD + 16 + (_j % 16), _j] = 1.0


def _stage1_body(q_ref, v_ref, wv_ref, bv_ref, wa_ref, ba_ref, wox_ref,
                 woy_ref, bx_ref, by_ref, sx_ref, sy_ref, wl_ref, hl_ref,
                 lsih_ref, g_ref, plo_ref, phi_ref, rpx_ref, rpy_ref,
                 val_ref, idx_ref, w_ref):
    b = pl.program_id(0)
    q = q_ref[0]
    v = v_ref[0]
    valf = jnp.dot(v, wv_ref[...], preferred_element_type=jnp.float32,
                   precision=lax.Precision.HIGHEST) + bv_ref[...]

    def bf16_bits(x):
        bits = lax.bitcast_convert_type(x, jnp.int32)
        rnd = bits + 0x7FFF + jnp.bitwise_and(
            lax.shift_right_logical(bits, 16), 1)
        return lax.shift_right_logical(rnd, 16)

    lo = jnp.dot(valf, plo_ref[...], preferred_element_type=jnp.float32,
                 precision=lax.Precision.HIGHEST)
    hi = jnp.dot(valf, phi_ref[...], preferred_element_type=jnp.float32,
                 precision=lax.Precision.HIGHEST)
    val_ref[0] = jnp.bitwise_or(bf16_bits(lo),
                                lax.shift_left(bf16_bits(hi), 16))
    logits = jnp.dot(q, wa_ref[...], preferred_element_type=jnp.float32, precision=lax.Precision.HIGHEST) + ba_ref[...]
    m = jnp.max(logits, axis=1, keepdims=True)
    e = jnp.exp(logits - m)
    den = jnp.dot(e, g_ref[...], preferred_element_type=jnp.float32, precision=lax.Precision.HIGHEST)
    att = e / den
    x = (jnp.dot(q, wox_ref[...], preferred_element_type=jnp.float32, precision=lax.Precision.HIGHEST)
         + jnp.dot(rpx_ref[0], sx_ref[...], preferred_element_type=jnp.float32, precision=lax.Precision.HIGHEST)
         + bx_ref[...])
    y = (jnp.dot(q, woy_ref[...], preferred_element_type=jnp.float32, precision=lax.Precision.HIGHEST)
         + jnp.dot(rpy_ref[0], sy_ref[...], preferred_element_type=jnp.float32, precision=lax.Precision.HIGHEST)
         + by_ref[...])
    wl = wl_ref[...]
    hl = hl_ref[...]
    x0 = jnp.floor(x)
    y0 = jnp.floor(y)
    fx1 = x - x0
    fx0 = 1.0 - fx1
    fy1 = y - y0
    fy0 = 1.0 - fy1
    base_b = b * (LQ * NH)
    lsih = lsih_ref[...]
    for t, (dx, dy) in enumerate(((0, 0), (1, 0), (0, 1), (1, 1))):
        xt = x0 + float(dx)
        yt = y0 + float(dy)
        valid = ((xt >= 0.0) & (xt <= wl - 1.0)
                 & (yt >= 0.0) & (yt <= hl - 1.0))
        xc = jnp.clip(xt, 0.0, wl - 1.0).astype(jnp.int32)
        yc = jnp.clip(yt, 0.0, hl - 1.0).astype(jnp.int32)
        spatial = yc * wl.astype(jnp.int32) + xc
        idx_ref[0, :, t * 128:(t + 1) * 128] = base_b + lsih + spatial * NH
        wt = (fx0 if dx == 0 else fx1) * (fy0 if dy == 0 else fy1)
        w_ref[0, :, t * 128:(t + 1) * 128] = att * wt * valid.astype(jnp.float32)


def _stage1(query, value, rpx, rpy, W_val, b_val, W_attn, b_attn,
            W_offx, W_offy, bx, by):
    consts = [
        jnp.asarray(_Sx), jnp.asarray(_Sy),
        jnp.asarray(_W_l).reshape(1, 128), jnp.asarray(_H_l).reshape(1, 128),
        jnp.asarray(_LSI_H).reshape(1, 128), jnp.asarray(_G),
        jnp.asarray(_Plo), jnp.asarray(_Phi),
    ]

    def whole(shape):
        return pl.BlockSpec(shape, lambda b, i: tuple(0 for _ in shape))

    return pl.pallas_call(
        _stage1_body,
        grid=(B, NBLK),
        in_specs=[
            pl.BlockSpec((1, BLK, D), lambda b, i: (b, i, 0)),
            pl.BlockSpec((1, BLK, D), lambda b, i: (b, i, 0)),
            whole((D, D)), whole((1, D)),
            whole((D, 128)), whole((1, 128)),
            whole((D, 128)), whole((D, 128)),
            whole((1, 128)), whole((1, 128)),
            whole((NL, 128)), whole((NL, 128)),
            whole((1, 128)), whole((1, 128)),
            whole((1, 128)), whole((128, 128)),
            whole((D, 128)), whole((D, 128)),
            pl.BlockSpec((1, BLK, NL), lambda b, i: (b, i, 0)),
            pl.BlockSpec((1, BLK, NL), lambda b, i: (b, i, 0)),
        ],
        out_specs=[
            pl.BlockSpec((1, BLK, 128), lambda b, i: (b, i, 0)),
            pl.BlockSpec((1, BLK, E), lambda b, i: (b, i, 0)),
            pl.BlockSpec((1, BLK, E), lambda b, i: (b, i, 0)),
        ],
        out_shape=[
            jax.ShapeDtypeStruct((B, LQ, 128), jnp.int32),
            jax.ShapeDtypeStruct((B, LQ, E), jnp.int32),
            jax.ShapeDtypeStruct((B, LQ, E), jnp.float32),
        ],
        compiler_params=pltpu.CompilerParams(
            dimension_semantics=("parallel", "parallel")),
    )(query, value, W_val, b_val, W_attn, b_attn, W_offx, W_offy, bx, by,
      *consts, rpx, rpy)


QC = 4
NQ = B * LQ
NGATH = E // 128


def _sc_body(tab_hbm, idx_hbm, w_hbm, out_hbm,
             idx_v, w_v, rows_v, out_v, gsem, osem):
    nc = 2
    wid = lax.axis_index("s") * nc + lax.axis_index("c")
    per_tile = NQ // 32
    iters = per_tile // QC
    qbase = wid * per_tile

    def fetch(q0, s):
        pltpu.sync_copy(idx_hbm.at[pl.ds(q0 * NGATH, QC * NGATH)], idx_v.at[s])
        pltpu.sync_copy(w_hbm.at[pl.ds(q0 * E, QC * E)], w_v.at[s])
        for j in range(QC * NGATH):
            pltpu.async_copy(
                tab_hbm.at[idx_v.at[s, j]],
                rows_v.at[s, pl.ds(j * 128, 128)], gsem)

    def drain(s):
        for j in range(QC * NGATH):
            pltpu.make_async_copy(
                tab_hbm.at[idx_v.at[s, j]],
                rows_v.at[s, pl.ds(j * 128, 128)], gsem).wait()

    def wait_out(s):
        pltpu.make_async_copy(
            out_v.at[s], out_hbm.at[pl.ds(0, QC * NH)], osem).wait()

    def compute(q0, s):
        def qh_body(qh, c2):
            rbase = (lax.shift_right_logical(qh, 3) * E
                     + jnp.bitwise_and(qh, 7) * 16)
            z = jnp.zeros((16,), jnp.float32)
            p0 = []
            p1 = []
            for t in range(NTAP):
                wvec = w_v[s, pl.ds(rbase + t * 128, 16)]
                u0, u1, v0, v1 = z, z, z, z
                for k in range(16):
                    wk = wvec[k]
                    r = rbase + t * 128 + k
                    row = rows_v[s, r, 0:16]
                    ev = lax.bitcast_convert_type(
                        lax.shift_left(row, 16), jnp.float32)
                    od = lax.bitcast_convert_type(row, jnp.float32)
                    if k & 1:
                        v0 = v0 + wk * ev
                        v1 = v1 + wk * od
                    else:
                        u0 = u0 + wk * ev
                        u1 = u1 + wk * od
                p0.append(u0 + v0)
                p1.append(u1 + v1)
            out_v[s, qh, 0:16] = (p0[0] + p0[1]) + (p0[2] + p0[3])
            out_v[s, qh, 16:32] = (p1[0] + p1[1]) + (p1[2] + p1[3])
            return c2

        lax.fori_loop(0, QC * NH, qh_body, 0)
        pltpu.async_copy(out_v.at[s], out_hbm.at[pl.ds(q0 * NH, QC * NH)], osem)

    npairs = iters // 2
    fetch(qbase, 0)

    def pair_body(p, carry):
        q0 = qbase + p * 2 * QC

        fetch(q0 + QC, 1)
        drain(0)

        @pl.when(p >= 1)
        def _():
            wait_out(0)

        compute(q0, 0)

        @pl.when(p + 1 < npairs)
        def _():
            fetch(q0 + 2 * QC, 0)

        drain(1)

        @pl.when(p >= 1)
        def _():
            wait_out(1)

        compute(q0 + QC, 1)
        return carry

    lax.fori_loop(0, npairs, pair_body, 0)
    wait_out(0)
    wait_out(1)


def _combine_sc(tab, idx4, w2):
    mesh = plsc.VectorSubcoreMesh(core_axis_name="c", subcore_axis_name="s")
    run = functools.partial(
        pl.kernel,
        mesh=mesh,
        out_type=jax.ShapeDtypeStruct((ROWS, HD), jnp.float32),
        name="msdeform_combine",
        scratch_types=[
            pltpu.VMEM((2, QC * NGATH, 128), jnp.int32),
            pltpu.VMEM((2, QC * E), jnp.float32),
            pltpu.VMEM((2, QC * E, HD // 2), jnp.int32),
            pltpu.VMEM((2, QC * NH, HD), jnp.float32),
            pltpu.SemaphoreType.DMA,
            pltpu.SemaphoreType.DMA,
        ],
        compiler_params=pltpu.CompilerParams(use_tc_tiling_on_sc=False),
    )(_sc_body)
    return run(tab, idx4, w2)


def _stage2_body(x_ref, w_ref, b_ref, o_ref):
    o_ref[0] = (jnp.dot(x_ref[0], w_ref[...], preferred_element_type=jnp.float32, precision=lax.Precision.HIGHEST)
                + b_ref[...])


def _stage2(x, W_out, b_out):
    return pl.pallas_call(
        _stage2_body,
        grid=(B, NBLK),
        in_specs=[
            pl.BlockSpec((1, BLK, D), lambda b, i: (b, i, 0)),
            pl.BlockSpec((D, D), lambda b, i: (0, 0)),
            pl.BlockSpec((1, D), lambda b, i: (0, 0)),
        ],
        out_specs=pl.BlockSpec((1, BLK, D), lambda b, i: (b, i, 0)),
        out_shape=jax.ShapeDtypeStruct((B, LQ, D), jnp.float32),
        compiler_params=pltpu.CompilerParams(
            dimension_semantics=("parallel", "parallel")),
    )(x, W_out, b_out)


_ATT_PERM = np.array([h * 16 + p * NL + l
                      for h in range(NH) for l in range(NL)
                      for p in range(NP)], np.int32)


def kernel(query, reference_points, value, spatial_shapes, level_start_index,
           W_off, b_off, W_attn, b_attn, W_val, b_val, W_out, b_out):
    W_attn = W_attn[:, _ATT_PERM]
    b_attn = b_attn[_ATT_PERM]
    rpx = reference_points[..., 0]
    rpy = reference_points[..., 1]
    W_offx = W_off[:, 0::2]
    W_offy = W_off[:, 1::2]
    bx = (b_off[0::2] - 0.5).reshape(1, 128)
    by = (b_off[1::2] - 0.5).reshape(1, 128)
    val, idxs, ws = _stage1(
        query, value, rpx, rpy, W_val, b_val.reshape(1, D),
        W_attn, b_attn.reshape(1, 128), W_offx, W_offy, bx, by)
    tab = val.reshape(ROWS, HD // 2)
    idx4 = idxs.reshape(NQ * NGATH, 128)
    w2 = ws.reshape(NQ * E)
    out1 = _combine_sc(tab, idx4, w2).reshape(B, LQ, D)
    return _stage2(out1, W_out, b_out.reshape(1, D))


# speedup vs baseline: 1.2758x; 1.0375x over previous
import functools

import jax
import jax.numpy as jnp
import numpy as np
from jax import lax
from jax.experimental import pallas as pl
from jax.experimental.pallas import tpu as pltpu
from jax.experimental.pallas import tpu_sc as plsc

D = 256
NH = 8
NL = 4
NP = 4
HD = D // NH
NTAP = 4
_SHAPES = [(128, 128), (64, 64), (32, 32), (16, 16)]
LQ = sum(h * w for h, w in _SHAPES)
B = 2
BLK = 640
NBLK = LQ // BLK
E = NH * NL * NP * NTAP
ROWS = B * LQ * NH

_lane = np.arange(128)
_lane_l = (_lane // 4) % 4
_lane_h = _lane // 16
_W_l = np.array([_SHAPES[l][1] for l in _lane_l], np.float32)
_H_l = np.array([_SHAPES[l][0] for l in _lane_l], np.float32)
_lsi_np = np.cumsum([0] + [h * w for h, w in _SHAPES])[:-1]
_LSI_H = (_lsi_np[_lane_l] * NH + _lane_h).astype(np.int32)
_Sx = np.zeros((NL, 128), np.float32)
_Sx[_lane_l, _lane] = _W_l
_Sy = np.zeros((NL, 128), np.float32)
_Sy[_lane_l, _lane] = _H_l
_G = (_lane_h[:, None] == _lane_h[None, :]).astype(np.float32)
_Plo = np.zeros((D, 128), np.float32)
_Phi = np.zeros((D, 128), np.float32)
for _j in range(128):
    _Plo[(_j // 16) * HD + (_j % 16), _j] = 1.0
    _Phi[(_j // 16) * HD + 16 + (_j % 16), _j] = 1.0


def _stage1_body(q_ref, v_ref, wv_ref, bv_ref, wa_ref, ba_ref, wox_ref,
                 woy_ref, bx_ref, by_ref, sx_ref, sy_ref, wl_ref, hl_ref,
                 lsih_ref, g_ref, plo_ref, phi_ref, rpx_ref, rpy_ref,
                 val_ref, idx_ref, w_ref):
    b = pl.program_id(0)
    q = q_ref[0]
    v = v_ref[0]
    valf = jnp.dot(v, wv_ref[...], preferred_element_type=jnp.float32,
                   precision=lax.Precision.HIGHEST) + bv_ref[...]

    def bf16_bits(x):
        bits = lax.bitcast_convert_type(x, jnp.int32)
        rnd = bits + 0x7FFF + jnp.bitwise_and(
            lax.shift_right_logical(bits, 16), 1)
        return lax.shift_right_logical(rnd, 16)

    lo = jnp.dot(valf, plo_ref[...], preferred_element_type=jnp.float32,
                 precision=lax.Precision.HIGHEST)
    hi = jnp.dot(valf, phi_ref[...], preferred_element_type=jnp.float32,
                 precision=lax.Precision.HIGHEST)
    val_ref[0] = jnp.bitwise_or(bf16_bits(lo),
                                lax.shift_left(bf16_bits(hi), 16))
    logits = jnp.dot(q, wa_ref[...], preferred_element_type=jnp.float32, precision=lax.Precision.HIGHEST) + ba_ref[...]
    m = jnp.max(logits, axis=1, keepdims=True)
    e = jnp.exp(logits - m)
    den = jnp.dot(e, g_ref[...], preferred_element_type=jnp.float32, precision=lax.Precision.HIGHEST)
    att = e / den
    x = (jnp.dot(q, wox_ref[...], preferred_element_type=jnp.float32, precision=lax.Precision.HIGHEST)
         + jnp.dot(rpx_ref[0], sx_ref[...], preferred_element_type=jnp.float32, precision=lax.Precision.HIGHEST)
         + bx_ref[...])
    y = (jnp.dot(q, woy_ref[...], preferred_element_type=jnp.float32, precision=lax.Precision.HIGHEST)
         + jnp.dot(rpy_ref[0], sy_ref[...], preferred_element_type=jnp.float32, precision=lax.Precision.HIGHEST)
         + by_ref[...])
    wl = wl_ref[...]
    hl = hl_ref[...]
    x0 = jnp.floor(x)
    y0 = jnp.floor(y)
    fx1 = x - x0
    fx0 = 1.0 - fx1
    fy1 = y - y0
    fy0 = 1.0 - fy1
    base_b = b * (LQ * NH)
    lsih = lsih_ref[...]
    for t, (dx, dy) in enumerate(((0, 0), (1, 0), (0, 1), (1, 1))):
        xt = x0 + float(dx)
        yt = y0 + float(dy)
        valid = ((xt >= 0.0) & (xt <= wl - 1.0)
                 & (yt >= 0.0) & (yt <= hl - 1.0))
        xc = jnp.clip(xt, 0.0, wl - 1.0).astype(jnp.int32)
        yc = jnp.clip(yt, 0.0, hl - 1.0).astype(jnp.int32)
        spatial = yc * wl.astype(jnp.int32) + xc
        idx_ref[0, :, t * 128:(t + 1) * 128] = base_b + lsih + spatial * NH
        wt = (fx0 if dx == 0 else fx1) * (fy0 if dy == 0 else fy1)
        w_ref[0, :, t * 128:(t + 1) * 128] = att * wt * valid.astype(jnp.float32)


def _stage1(query, value, rpx, rpy, W_val, b_val, W_attn, b_attn,
            W_offx, W_offy, bx, by):
    consts = [
        jnp.asarray(_Sx), jnp.asarray(_Sy),
        jnp.asarray(_W_l).reshape(1, 128), jnp.asarray(_H_l).reshape(1, 128),
        jnp.asarray(_LSI_H).reshape(1, 128), jnp.asarray(_G),
        jnp.asarray(_Plo), jnp.asarray(_Phi),
    ]

    def whole(shape):
        return pl.BlockSpec(shape, lambda b, i: tuple(0 for _ in shape))

    return pl.pallas_call(
        _stage1_body,
        grid=(B, NBLK),
        in_specs=[
            pl.BlockSpec((1, BLK, D), lambda b, i: (b, i, 0)),
            pl.BlockSpec((1, BLK, D), lambda b, i: (b, i, 0)),
            whole((D, D)), whole((1, D)),
            whole((D, 128)), whole((1, 128)),
            whole((D, 128)), whole((D, 128)),
            whole((1, 128)), whole((1, 128)),
            whole((NL, 128)), whole((NL, 128)),
            whole((1, 128)), whole((1, 128)),
            whole((1, 128)), whole((128, 128)),
            whole((D, 128)), whole((D, 128)),
            pl.BlockSpec((1, BLK, NL), lambda b, i: (b, i, 0)),
            pl.BlockSpec((1, BLK, NL), lambda b, i: (b, i, 0)),
        ],
        out_specs=[
            pl.BlockSpec((1, BLK, 128), lambda b, i: (b, i, 0)),
            pl.BlockSpec((1, BLK, E), lambda b, i: (b, i, 0)),
            pl.BlockSpec((1, BLK, E), lambda b, i: (b, i, 0)),
        ],
        out_shape=[
            jax.ShapeDtypeStruct((B, LQ, 128), jnp.int32),
            jax.ShapeDtypeStruct((B, LQ, E), jnp.int32),
            jax.ShapeDtypeStruct((B, LQ, E), jnp.float32),
        ],
        compiler_params=pltpu.CompilerParams(
            dimension_semantics=("parallel", "parallel")),
    )(query, value, W_val, b_val, W_attn, b_attn, W_offx, W_offy, bx, by,
      *consts, rpx, rpy)


QC = 5
NQ = B * LQ
NGATH = E // 128


def _sc_body(tab_hbm, idx_hbm, w_hbm, out_hbm,
             idx_v, w_v, rows_v, out_v, gsem, osem):
    nc = 2
    wid = lax.axis_index("s") * nc + lax.axis_index("c")
    per_tile = NQ // 32
    iters = per_tile // QC
    qbase = wid * per_tile

    def fetch(q0, s):
        pltpu.sync_copy(idx_hbm.at[pl.ds(q0 * NGATH, QC * NGATH)], idx_v.at[s])
        pltpu.sync_copy(w_hbm.at[pl.ds(q0 * E, QC * E)], w_v.at[s])
        for j in range(QC * NGATH):
            pltpu.async_copy(
                tab_hbm.at[idx_v.at[s, j]],
                rows_v.at[s, pl.ds(j * 128, 128)], gsem)

    def drain(s):
        for j in range(QC * NGATH):
            pltpu.make_async_copy(
                tab_hbm.at[idx_v.at[s, j]],
                rows_v.at[s, pl.ds(j * 128, 128)], gsem).wait()

    def wait_out(s):
        pltpu.make_async_copy(
            out_v.at[s], out_hbm.at[pl.ds(0, QC * NH)], osem).wait()

    def compute(q0, s):
        def qh_body(qh, c2):
            rbase = (lax.shift_right_logical(qh, 3) * E
                     + jnp.bitwise_and(qh, 7) * 16)
            z = jnp.zeros((16,), jnp.float32)
            p0 = []
            p1 = []
            for t in range(NTAP):
                wvec = w_v[s, pl.ds(rbase + t * 128, 16)]
                u0, u1, v0, v1 = z, z, z, z
                for k in range(16):
                    wk = wvec[k]
                    r = rbase + t * 128 + k
                    row = rows_v[s, r, 0:16]
                    ev = lax.bitcast_convert_type(
                        lax.shift_left(row, 16), jnp.float32)
                    od = lax.bitcast_convert_type(row, jnp.float32)
                    if k & 1:
                        v0 = v0 + wk * ev
                        v1 = v1 + wk * od
                    else:
                        u0 = u0 + wk * ev
                        u1 = u1 + wk * od
                p0.append(u0 + v0)
                p1.append(u1 + v1)
            out_v[s, qh, 0:16] = (p0[0] + p0[1]) + (p0[2] + p0[3])
            out_v[s, qh, 16:32] = (p1[0] + p1[1]) + (p1[2] + p1[3])
            return c2

        lax.fori_loop(0, QC * NH, qh_body, 0)
        pltpu.async_copy(out_v.at[s], out_hbm.at[pl.ds(q0 * NH, QC * NH)], osem)

    npairs = iters // 2
    fetch(qbase, 0)

    def pair_body(p, carry):
        q0 = qbase + p * 2 * QC

        fetch(q0 + QC, 1)
        drain(0)

        @pl.when(p >= 1)
        def _():
            wait_out(0)

        compute(q0, 0)

        @pl.when(p + 1 < npairs)
        def _():
            fetch(q0 + 2 * QC, 0)

        drain(1)

        @pl.when(p >= 1)
        def _():
            wait_out(1)

        compute(q0 + QC, 1)
        return carry

    lax.fori_loop(0, npairs, pair_body, 0)
    wait_out(0)
    wait_out(1)


def _combine_sc(tab, idx4, w2):
    mesh = plsc.VectorSubcoreMesh(core_axis_name="c", subcore_axis_name="s")
    run = functools.partial(
        pl.kernel,
        mesh=mesh,
        out_type=jax.ShapeDtypeStruct((ROWS, HD), jnp.float32),
        name="msdeform_combine",
        scratch_types=[
            pltpu.VMEM((2, QC * NGATH, 128), jnp.int32),
            pltpu.VMEM((2, QC * E), jnp.float32),
            pltpu.VMEM((2, QC * E, HD // 2), jnp.int32),
            pltpu.VMEM((2, QC * NH, HD), jnp.float32),
            pltpu.SemaphoreType.DMA,
            pltpu.SemaphoreType.DMA,
        ],
        compiler_params=pltpu.CompilerParams(use_tc_tiling_on_sc=False),
    )(_sc_body)
    return run(tab, idx4, w2)


def _stage2_body(x_ref, w_ref, b_ref, o_ref):
    o_ref[0] = (jnp.dot(x_ref[0], w_ref[...], preferred_element_type=jnp.float32, precision=lax.Precision.HIGHEST)
                + b_ref[...])


def _stage2(x, W_out, b_out):
    return pl.pallas_call(
        _stage2_body,
        grid=(B, NBLK),
        in_specs=[
            pl.BlockSpec((1, BLK, D), lambda b, i: (b, i, 0)),
            pl.BlockSpec((D, D), lambda b, i: (0, 0)),
            pl.BlockSpec((1, D), lambda b, i: (0, 0)),
        ],
        out_specs=pl.BlockSpec((1, BLK, D), lambda b, i: (b, i, 0)),
        out_shape=jax.ShapeDtypeStruct((B, LQ, D), jnp.float32),
        compiler_params=pltpu.CompilerParams(
            dimension_semantics=("parallel", "parallel")),
    )(x, W_out, b_out)


_ATT_PERM = np.array([h * 16 + p * NL + l
                      for h in range(NH) for l in range(NL)
                      for p in range(NP)], np.int32)


def kernel(query, reference_points, value, spatial_shapes, level_start_index,
           W_off, b_off, W_attn, b_attn, W_val, b_val, W_out, b_out):
    W_attn = W_attn[:, _ATT_PERM]
    b_attn = b_attn[_ATT_PERM]
    rpx = reference_points[..., 0]
    rpy = reference_points[..., 1]
    W_offx = W_off[:, 0::2]
    W_offy = W_off[:, 1::2]
    bx = (b_off[0::2] - 0.5).reshape(1, 128)
    by = (b_off[1::2] - 0.5).reshape(1, 128)
    val, idxs, ws = _stage1(
        query, value, rpx, rpy, W_val, b_val.reshape(1, D),
        W_attn, b_attn.reshape(1, 128), W_offx, W_offy, bx, by)
    tab = val.reshape(ROWS, HD // 2)
    idx4 = idxs.reshape(NQ * NGATH, 128)
    w2 = ws.reshape(NQ * E)
    out1 = _combine_sc(tab, idx4, w2).reshape(B, LQ, D)
    return _stage2(out1, W_out, b_out.reshape(1, D))


# speedup vs baseline: 1.5242x; 1.1948x over previous
import functools

import jax
import jax.numpy as jnp
import numpy as np
from jax import lax
from jax.experimental import pallas as pl
from jax.experimental.pallas import tpu as pltpu
from jax.experimental.pallas import tpu_sc as plsc

D = 256
NH = 8
NL = 4
NP = 4
HD = D // NH
NTAP = 4
_SHAPES = [(128, 128), (64, 64), (32, 32), (16, 16)]
LQ = sum(h * w for h, w in _SHAPES)
B = 2
BLK = 640
NBLK = LQ // BLK
E = NH * NL * NP * NTAP
ROWS = B * LQ * NH

_lane = np.arange(128)
_lane_l = (_lane // 4) % 4
_lane_h = _lane // 16
_W_l = np.array([_SHAPES[l][1] for l in _lane_l], np.float32)
_H_l = np.array([_SHAPES[l][0] for l in _lane_l], np.float32)
_lsi_np = np.cumsum([0] + [h * w for h, w in _SHAPES])[:-1]
_LSI_H = (_lsi_np[_lane_l] * NH + _lane_h).astype(np.int32)
_Sx = np.zeros((NL, 128), np.float32)
_Sx[_lane_l, _lane] = _W_l
_Sy = np.zeros((NL, 128), np.float32)
_Sy[_lane_l, _lane] = _H_l
_G = (_lane_h[:, None] == _lane_h[None, :]).astype(np.float32)
_Plo = np.zeros((D, 128), np.float32)
_Phi = np.zeros((D, 128), np.float32)
for _j in range(128):
    _Plo[(_j // 16) * HD + (_j % 16), _j] = 1.0
    _Phi[(_j // 16) * HD + 16 + (_j % 16), _j] = 1.0


def _stage1_body(q_ref, v_ref, wv_ref, bv_ref, wa_ref, ba_ref, wox_ref,
                 woy_ref, bx_ref, by_ref, sx_ref, sy_ref, wl_ref, hl_ref,
                 lsih_ref, g_ref, plo_ref, phi_ref, rpx_ref, rpy_ref,
                 val_ref, idx_ref, w_ref):
    b = pl.program_id(0)
    q = q_ref[0]
    v = v_ref[0]
    valf = jnp.dot(v, wv_ref[...], preferred_element_type=jnp.float32,
                   precision=lax.Precision.HIGHEST) + bv_ref[...]

    def bf16_bits(x):
        bits = lax.bitcast_convert_type(x, jnp.int32)
        rnd = bits + 0x7FFF + jnp.bitwise_and(
            lax.shift_right_logical(bits, 16), 1)
        return lax.shift_right_logical(rnd, 16)

    lo = jnp.dot(valf, plo_ref[...], preferred_element_type=jnp.float32,
                 precision=lax.Precision.HIGHEST)
    hi = jnp.dot(valf, phi_ref[...], preferred_element_type=jnp.float32,
                 precision=lax.Precision.HIGHEST)
    val_ref[0] = jnp.bitwise_or(bf16_bits(lo),
                                lax.shift_left(bf16_bits(hi), 16))
    logits = jnp.dot(q, wa_ref[...], preferred_element_type=jnp.float32, precision=lax.Precision.HIGHEST) + ba_ref[...]
    m = jnp.max(logits, axis=1, keepdims=True)
    e = jnp.exp(logits - m)
    den = jnp.dot(e, g_ref[...], preferred_element_type=jnp.float32, precision=lax.Precision.HIGHEST)
    att = e / den
    x = (jnp.dot(q, wox_ref[...], preferred_element_type=jnp.float32, precision=lax.Precision.HIGHEST)
         + jnp.dot(rpx_ref[0], sx_ref[...], preferred_element_type=jnp.float32, precision=lax.Precision.HIGHEST)
         + bx_ref[...])
    y = (jnp.dot(q, woy_ref[...], preferred_element_type=jnp.float32, precision=lax.Precision.HIGHEST)
         + jnp.dot(rpy_ref[0], sy_ref[...], preferred_element_type=jnp.float32, precision=lax.Precision.HIGHEST)
         + by_ref[...])
    wl = wl_ref[...]
    hl = hl_ref[...]
    x0 = jnp.floor(x)
    y0 = jnp.floor(y)
    fx1 = x - x0
    fx0 = 1.0 - fx1
    fy1 = y - y0
    fy0 = 1.0 - fy1
    base_b = b * (LQ * NH)
    lsih = lsih_ref[...]
    for t, (dx, dy) in enumerate(((0, 0), (1, 0), (0, 1), (1, 1))):
        xt = x0 + float(dx)
        yt = y0 + float(dy)
        valid = ((xt >= 0.0) & (xt <= wl - 1.0)
                 & (yt >= 0.0) & (yt <= hl - 1.0))
        xc = jnp.clip(xt, 0.0, wl - 1.0).astype(jnp.int32)
        yc = jnp.clip(yt, 0.0, hl - 1.0).astype(jnp.int32)
        spatial = yc * wl.astype(jnp.int32) + xc
        idx_ref[0, :, t * 128:(t + 1) * 128] = base_b + lsih + spatial * NH
        wt = (fx0 if dx == 0 else fx1) * (fy0 if dy == 0 else fy1)
        w_ref[0, :, t * 128:(t + 1) * 128] = att * wt * valid.astype(jnp.float32)


def _stage1(query, value, rpx, rpy, W_val, b_val, W_attn, b_attn,
            W_offx, W_offy, bx, by):
    consts = [
        jnp.asarray(_Sx), jnp.asarray(_Sy),
        jnp.asarray(_W_l).reshape(1, 128), jnp.asarray(_H_l).reshape(1, 128),
        jnp.asarray(_LSI_H).reshape(1, 128), jnp.asarray(_G),
        jnp.asarray(_Plo), jnp.asarray(_Phi),
    ]

    def whole(shape):
        return pl.BlockSpec(shape, lambda b, i: tuple(0 for _ in shape))

    return pl.pallas_call(
        _stage1_body,
        grid=(B, NBLK),
        in_specs=[
            pl.BlockSpec((1, BLK, D), lambda b, i: (b, i, 0)),
            pl.BlockSpec((1, BLK, D), lambda b, i: (b, i, 0)),
            whole((D, D)), whole((1, D)),
            whole((D, 128)), whole((1, 128)),
            whole((D, 128)), whole((D, 128)),
            whole((1, 128)), whole((1, 128)),
            whole((NL, 128)), whole((NL, 128)),
            whole((1, 128)), whole((1, 128)),
            whole((1, 128)), whole((128, 128)),
            whole((D, 128)), whole((D, 128)),
            pl.BlockSpec((1, BLK, NL), lambda b, i: (b, i, 0)),
            pl.BlockSpec((1, BLK, NL), lambda b, i: (b, i, 0)),
        ],
        out_specs=[
            pl.BlockSpec((1, BLK, 128), lambda b, i: (b, i, 0)),
            pl.BlockSpec((1, BLK, E), lambda b, i: (b, i, 0)),
            pl.BlockSpec((1, BLK, E), lambda b, i: (b, i, 0)),
        ],
        out_shape=[
            jax.ShapeDtypeStruct((B, LQ, 128), jnp.int32),
            jax.ShapeDtypeStruct((B, LQ, E), jnp.int32),
            jax.ShapeDtypeStruct((B, LQ, E), jnp.float32),
        ],
        compiler_params=pltpu.CompilerParams(
            dimension_semantics=("parallel", "parallel")),
    )(query, value, W_val, b_val, W_attn, b_attn, W_offx, W_offy, bx, by,
      *consts, rpx, rpy)


QC = 5
NQ = B * LQ
NGATH = E // 128


def _sc_body(tab_hbm, idx_hbm, w_hbm, out_hbm,
             idx_v, w_v, rows_v, out_v, gsem, osem):
    nc = 2
    wid = lax.axis_index("s") * nc + lax.axis_index("c")
    per_tile = NQ // 32
    iters = per_tile // QC
    qbase = wid * per_tile

    def fetch(q0, s):
        pltpu.sync_copy(idx_hbm.at[pl.ds(q0 * NGATH, QC * NGATH)], idx_v.at[s])
        pltpu.sync_copy(w_hbm.at[pl.ds(q0 * E, QC * E)], w_v.at[s])
        for j in range(QC * NGATH):
            pltpu.async_copy(
                tab_hbm.at[idx_v.at[s, j]],
                rows_v.at[s, pl.ds(j * 128, 128)], gsem)

    def drain(s):
        for j in range(QC * NGATH):
            pltpu.make_async_copy(
                tab_hbm.at[idx_v.at[s, j]],
                rows_v.at[s, pl.ds(j * 128, 128)], gsem).wait()

    def wait_out(s):
        pltpu.make_async_copy(
            out_v.at[s], out_hbm.at[pl.ds(0, QC * NH)], osem).wait()

    def compute(q0, s):
        @functools.partial(plsc.parallel_loop, 0, QC * NH, unroll=2)
        def qh_body(qh):
            rbase = (lax.shift_right_logical(qh, 3) * E
                     + jnp.bitwise_and(qh, 7) * 16)
            z = jnp.zeros((16,), jnp.float32)
            p0 = []
            p1 = []
            for t in range(NTAP):
                wvec = w_v[s, pl.ds(rbase + t * 128, 16)]
                u0, u1, v0, v1 = z, z, z, z
                for k in range(16):
                    wk = wvec[k]
                    r = rbase + t * 128 + k
                    row = rows_v[s, r, 0:16]
                    ev = lax.bitcast_convert_type(
                        lax.shift_left(row, 16), jnp.float32)
                    od = lax.bitcast_convert_type(row, jnp.float32)
                    if k & 1:
                        v0 = v0 + wk * ev
                        v1 = v1 + wk * od
                    else:
                        u0 = u0 + wk * ev
                        u1 = u1 + wk * od
                p0.append(u0 + v0)
                p1.append(u1 + v1)
            out_v[s, qh, 0:16] = (p0[0] + p0[1]) + (p0[2] + p0[3])
            out_v[s, qh, 16:32] = (p1[0] + p1[1]) + (p1[2] + p1[3])

        pltpu.async_copy(out_v.at[s], out_hbm.at[pl.ds(q0 * NH, QC * NH)], osem)

    npairs = iters // 2
    fetch(qbase, 0)

    def pair_body(p, carry):
        q0 = qbase + p * 2 * QC

        fetch(q0 + QC, 1)
        drain(0)

        @pl.when(p >= 1)
        def _():
            wait_out(0)

        compute(q0, 0)

        @pl.when(p + 1 < npairs)
        def _():
            fetch(q0 + 2 * QC, 0)

        drain(1)

        @pl.when(p >= 1)
        def _():
            wait_out(1)

        compute(q0 + QC, 1)
        return carry

    lax.fori_loop(0, npairs, pair_body, 0)
    wait_out(0)
    wait_out(1)


def _combine_sc(tab, idx4, w2):
    mesh = plsc.VectorSubcoreMesh(core_axis_name="c", subcore_axis_name="s")
    run = functools.partial(
        pl.kernel,
        mesh=mesh,
        out_type=jax.ShapeDtypeStruct((ROWS, HD), jnp.float32),
        name="msdeform_combine",
        scratch_types=[
            pltpu.VMEM((2, QC * NGATH, 128), jnp.int32),
            pltpu.VMEM((2, QC * E), jnp.float32),
            pltpu.VMEM((2, QC * E, HD // 2), jnp.int32),
            pltpu.VMEM((2, QC * NH, HD), jnp.float32),
            pltpu.SemaphoreType.DMA,
            pltpu.SemaphoreType.DMA,
        ],
        compiler_params=pltpu.CompilerParams(use_tc_tiling_on_sc=False),
    )(_sc_body)
    return run(tab, idx4, w2)


def _stage2_body(x_ref, w_ref, b_ref, o_ref):
    o_ref[0] = (jnp.dot(x_ref[0], w_ref[...], preferred_element_type=jnp.float32, precision=lax.Precision.HIGHEST)
                + b_ref[...])


def _stage2(x, W_out, b_out):
    return pl.pallas_call(
        _stage2_body,
        grid=(B, NBLK),
        in_specs=[
            pl.BlockSpec((1, BLK, D), lambda b, i: (b, i, 0)),
            pl.BlockSpec((D, D), lambda b, i: (0, 0)),
            pl.BlockSpec((1, D), lambda b, i: (0, 0)),
        ],
        out_specs=pl.BlockSpec((1, BLK, D), lambda b, i: (b, i, 0)),
        out_shape=jax.ShapeDtypeStruct((B, LQ, D), jnp.float32),
        compiler_params=pltpu.CompilerParams(
            dimension_semantics=("parallel", "parallel")),
    )(x, W_out, b_out)


_ATT_PERM = np.array([h * 16 + p * NL + l
                      for h in range(NH) for l in range(NL)
                      for p in range(NP)], np.int32)


def kernel(query, reference_points, value, spatial_shapes, level_start_index,
           W_off, b_off, W_attn, b_attn, W_val, b_val, W_out, b_out):
    W_attn = W_attn[:, _ATT_PERM]
    b_attn = b_attn[_ATT_PERM]
    rpx = reference_points[..., 0]
    rpy = reference_points[..., 1]
    W_offx = W_off[:, 0::2]
    W_offy = W_off[:, 1::2]
    bx = (b_off[0::2] - 0.5).reshape(1, 128)
    by = (b_off[1::2] - 0.5).reshape(1, 128)
    val, idxs, ws = _stage1(
        query, value, rpx, rpy, W_val, b_val.reshape(1, D),
        W_attn, b_attn.reshape(1, 128), W_offx, W_offy, bx, by)
    tab = val.reshape(ROWS, HD // 2)
    idx4 = idxs.reshape(NQ * NGATH, 128)
    w2 = ws.reshape(NQ * E)
    out1 = _combine_sc(tab, idx4, w2).reshape(B, LQ, D)
    return _stage2(out1, W_out, b_out.reshape(1, D))
